# baseline (speedup 1.0000x reference)
import functools

import jax
import jax.numpy as jnp
from jax import lax
from jax.experimental import pallas as pl
from jax.experimental.pallas import tpu as pltpu
from jax.experimental.pallas import tpu_sc as plsc

HIDDEN = 768
PAD = 1
EPS = 1e-5

NC, NS, L = 2, 16, 16
NW = NC * NS
B = 4 * 2048
SEQ = 2048
ROWS_PER_W = B // NW
CHUNK = 32
NCHUNK = ROWS_PER_W // CHUNK
NG = HIDDEN // L


def _vrsqrt(v):
    i = plsc.bitcast(v, jnp.int32)
    i = jnp.int32(0x5F3759DF) - (i >> 1)
    y = plsc.bitcast(i, jnp.float32)
    for _ in range(3):
        y = y * (1.5 - 0.5 * v * y * y)
    return y


def _sc_body(ids_hbm, wtab_hbm, ptab_hbm, tt_hbm, gamma_hbm, beta_hbm,
             out_hbm,
             ids_v, wida, pida, wbuf, pbuf, tt_v, gam_v, bet_v, pref_v,
             wsem, psem):
    c = lax.axis_index("c")
    s = lax.axis_index("s")
    wid = c * NS + s
    chk = wid % 8
    base_tok = wid * ROWS_PER_W
    seq_tok = base_tok - chk * ROWS_PER_W

    pltpu.sync_copy(ids_hbm.at[pl.ds(base_tok, ROWS_PER_W)], ids_v)
    pltpu.sync_copy(gamma_hbm, gam_v)
    pltpu.sync_copy(beta_hbm, bet_v)
    pltpu.sync_copy(tt_hbm.at[0], tt_v)

    def jbody(j, acc):
        pltpu.sync_copy(ids_hbm.at[pl.ds(seq_tok + j * ROWS_PER_W,
                                         ROWS_PER_W)], pref_v)

        def cbody(i, a):
            v = pref_v[pl.ds(i * L, L)]
            return a + jnp.where(v != PAD, 1, 0).astype(jnp.int32)

        return lax.fori_loop(0, ROWS_PER_W // L, cbody, acc)

    off = lax.fori_loop(0, chk, jbody, jnp.zeros((L,), jnp.int32))
    off = jnp.full((L,), jnp.sum(off), jnp.int32)

    gpc = CHUNK // L

    def pbody(i, run):
        v = ids_v[pl.ds(i * L, L)]
        m = jnp.where(v != PAD, 1, 0).astype(jnp.int32)
        cs = plsc.cumsum(m)
        k = i // gpc
        o = (i % gpc) * L
        wida[k, pl.ds(o, L)] = v
        pida[k, pl.ds(o, L)] = (cs + run) * m + PAD
        return run + jnp.sum(m)

    lax.fori_loop(0, ROWS_PER_W // L, pbody, off)

    inv_h = jnp.float32(1.0 / HIDDEN)
    for k in range(NCHUNK):
        cw = pltpu.async_copy(wtab_hbm.at[wida.at[k]], wbuf, wsem)
        cp = pltpu.async_copy(ptab_hbm.at[pida.at[k]], pbuf, psem)
        cw.wait()
        cp.wait()

        def rbody(r, _):
            def g1(g, carry):
                sacc, qacc = carry
                x = (wbuf[r, pl.ds(g * L, L)] + pbuf[r, pl.ds(g * L, L)]
                     + tt_v[pl.ds(g * L, L)])
                wbuf[r, pl.ds(g * L, L)] = x
                return sacc + x, qacc + x * x

            z = jnp.zeros((L,), jnp.float32)
            sacc, qacc = lax.fori_loop(0, NG, g1, (z, z))
            mean = jnp.sum(sacc) * inv_h
            ex2 = jnp.sum(qacc) * inv_h
            mean_v = jnp.full((L,), mean, jnp.float32)
            var_v = jnp.full((L,), ex2, jnp.float32) - mean_v * mean_v + EPS
            rinv_v = _vrsqrt(var_v)

            def g2(g, _):
                x = wbuf[r, pl.ds(g * L, L)]
                y = ((x - mean_v) * rinv_v * gam_v[pl.ds(g * L, L)]
                     + bet_v[pl.ds(g * L, L)])
                wbuf[r, pl.ds(g * L, L)] = y
                return 0

            lax.fori_loop(0, NG, g2, 0)
            return 0

        lax.fori_loop(0, CHUNK, rbody, 0)
        pltpu.sync_copy(wbuf, out_hbm.at[pl.ds(base_tok + k * CHUNK, CHUNK)])


@jax.jit
def _run(ids, wtab, ptab, tttab, gamma, beta):
    mesh = plsc.VectorSubcoreMesh(
        core_axis_name="c", subcore_axis_name="s",
        num_cores=NC, num_subcores=NS)
    f = pl.kernel(
        _sc_body,
        out_type=jax.ShapeDtypeStruct((B, HIDDEN), jnp.float32),
        mesh=mesh,
        compiler_params=pltpu.CompilerParams(needs_layout_passes=False),
        scratch_types=[
            pltpu.VMEM((ROWS_PER_W,), jnp.int32),
            pltpu.VMEM((NCHUNK, CHUNK), jnp.int32),
            pltpu.VMEM((NCHUNK, CHUNK), jnp.int32),
            pltpu.VMEM((CHUNK, HIDDEN), jnp.float32),
            pltpu.VMEM((CHUNK, HIDDEN), jnp.float32),
            pltpu.VMEM((HIDDEN,), jnp.float32),
            pltpu.VMEM((HIDDEN,), jnp.float32),
            pltpu.VMEM((HIDDEN,), jnp.float32),
            pltpu.VMEM((ROWS_PER_W,), jnp.int32),
            pltpu.SemaphoreType.DMA,
            pltpu.SemaphoreType.DMA,
        ],
    )
    return f(ids, wtab, ptab, tttab, gamma, beta)


def kernel(input_ids, word_embeddings, position_embeddings,
           token_type_embeddings, ln_gamma, ln_beta):
    ids = input_ids.reshape(-1).astype(jnp.int32)
    out = _run(ids, word_embeddings, position_embeddings,
               token_type_embeddings, ln_gamma, ln_beta)
    return out.reshape(input_ids.shape + (HIDDEN,))


# speedup vs baseline: 1.9074x; 1.9074x over previous
import functools

import jax
import jax.numpy as jnp
from jax import lax
from jax.experimental import pallas as pl
from jax.experimental.pallas import tpu as pltpu
from jax.experimental.pallas import tpu_sc as plsc

HIDDEN = 768
PAD = 1
EPS = 1e-5
MAX_POS = 2050

NC, NS, L = 2, 16, 16
NW = NC * NS
B = 4 * 2048
ROWS_PER_W = B // NW
CHUNK = 16
NCHUNK = ROWS_PER_W // CHUNK
NG = HIDDEN // L


def _gs(g):
    return pl.ds(g * L, L)


def _vrsqrt(v):
    i = plsc.bitcast(v, jnp.int32)
    i = jnp.int32(0x5F3759DF) - (i >> 1)
    y = plsc.bitcast(i, jnp.float32)
    for _ in range(3):
        y = y * (1.5 - 0.5 * v * y * y)
    return y


def _fold_body(pos_ref, tt_ref, o_ref):
    o_ref[...] = pos_ref[...] + tt_ref[0:1, :]


@jax.jit
def _fold_tt(ptab, tttab):
    return pl.pallas_call(
        _fold_body,
        out_shape=jax.ShapeDtypeStruct((MAX_POS, HIDDEN), jnp.float32),
    )(ptab, tttab)


def _sc_body(ids_hbm, wtab_hbm, ptt_hbm, gamma_hbm, beta_hbm,
             out_hbm,
             ids_v, wida, pida, gam_v, bet_v, pref_v,
             wbuf0, wbuf1, pbuf0, pbuf1, obuf0, obuf1,
             wsem0, wsem1, psem0, psem1, osem0, osem1):
    c = lax.axis_index("c")
    s = lax.axis_index("s")
    wid = c * NS + s
    chk = wid % 8
    base_tok = wid * ROWS_PER_W
    seq_tok = base_tok - chk * ROWS_PER_W

    pltpu.sync_copy(ids_hbm.at[pl.ds(base_tok, ROWS_PER_W)], ids_v)
    pltpu.sync_copy(gamma_hbm, gam_v)
    pltpu.sync_copy(beta_hbm, bet_v)

    def jbody(j, acc):
        pltpu.sync_copy(ids_hbm.at[pl.ds(seq_tok + j * ROWS_PER_W,
                                         ROWS_PER_W)], pref_v)

        def cbody(i, a):
            v = pref_v[pl.ds(i * L, L)]
            return a + jnp.where(v != PAD, 1, 0).astype(jnp.int32)

        return lax.fori_loop(0, ROWS_PER_W // L, cbody, acc)

    off = lax.fori_loop(0, chk, jbody, jnp.zeros((L,), jnp.int32))
    off = jnp.full((L,), jnp.sum(off), jnp.int32)

    def pbody(i, run):
        v = ids_v[pl.ds(i * L, L)]
        m = jnp.where(v != PAD, 1, 0).astype(jnp.int32)
        cs = plsc.cumsum(m)
        wida[i] = v
        pida[i] = (cs + run) * m + PAD
        return run + jnp.sum(m)

    lax.fori_loop(0, NCHUNK, pbody, off)

    wbufs = (wbuf0, wbuf1)
    pbufs = (pbuf0, pbuf1)
    obufs = (obuf0, obuf1)
    wsems = (wsem0, wsem1)
    psems = (psem0, psem1)
    osems = (osem0, osem1)
    inv_h = jnp.float32(1.0 / HIDDEN)
    z = jnp.zeros((L,), jnp.float32)

    def chunk_step(k, b, first, last):
        wb, pb, ob = wbufs[b], pbufs[b], obufs[b]
        if not first:
            pltpu.make_async_copy(ob, out_hbm.at[pl.ds(base_tok, CHUNK)],
                                  osems[b]).wait()
        pltpu.make_async_copy(wtab_hbm.at[wida.at[k]], wb, wsems[b]).wait()
        pltpu.make_async_copy(ptt_hbm.at[pida.at[k]], pb, psems[b]).wait()

        def rbody(r, _):
            s0 = s1 = s2 = z
            q0 = q1 = q2 = z
            xs = []
            for g in range(NG):
                x = wb[r, _gs(g)] + pb[r, _gs(g)]
                pb[r, _gs(g)] = x
                if g % 3 == 0:
                    s0 = s0 + x
                    q0 = q0 + x * x
                elif g % 3 == 1:
                    s1 = s1 + x
                    q1 = q1 + x * x
                else:
                    s2 = s2 + x
                    q2 = q2 + x * x
            sacc = s0 + s1 + s2
            qacc = q0 + q1 + q2
            mean = jnp.sum(sacc) * inv_h
            ex2 = jnp.sum(qacc) * inv_h
            mean_v = jnp.full((L,), mean, jnp.float32)
            var_v = jnp.full((L,), ex2, jnp.float32) - mean_v * mean_v + EPS
            rinv_v = _vrsqrt(var_v)
            mb_v = mean_v * rinv_v
            for g in range(NG):
                x = pb[r, _gs(g)]
                t = x * rinv_v - mb_v
                ob[r, _gs(g)] = t * gam_v[_gs(g)] + bet_v[_gs(g)]
            return 0

        lax.fori_loop(0, CHUNK, rbody, 0)
        pltpu.async_copy(ob, out_hbm.at[pl.ds(base_tok + k * CHUNK, CHUNK)],
                         osems[b])
        if not last:
            k2 = k + 2
            pltpu.async_copy(wtab_hbm.at[wida.at[k2]], wb, wsems[b])
            pltpu.async_copy(ptt_hbm.at[pida.at[k2]], pb, psems[b])

    pltpu.async_copy(wtab_hbm.at[wida.at[0]], wbuf0, wsem0)
    pltpu.async_copy(ptt_hbm.at[pida.at[0]], pbuf0, psem0)
    pltpu.async_copy(wtab_hbm.at[wida.at[1]], wbuf1, wsem1)
    pltpu.async_copy(ptt_hbm.at[pida.at[1]], pbuf1, psem1)

    chunk_step(0, 0, True, False)
    chunk_step(1, 1, True, False)

    def loop_body(i, _):
        k = 2 * i + 2
        chunk_step(k, 0, False, False)
        chunk_step(k + 1, 1, False, False)
        return 0

    lax.fori_loop(0, (NCHUNK - 4) // 2, loop_body, 0)

    chunk_step(NCHUNK - 2, 0, False, True)
    chunk_step(NCHUNK - 1, 1, False, True)

    pltpu.make_async_copy(obuf0, out_hbm.at[pl.ds(base_tok, CHUNK)],
                          osem0).wait()
    pltpu.make_async_copy(obuf1, out_hbm.at[pl.ds(base_tok, CHUNK)],
                          osem1).wait()


@jax.jit
def _run(ids, wtab, ptt, gamma, beta):
    mesh = plsc.VectorSubcoreMesh(
        core_axis_name="c", subcore_axis_name="s",
        num_cores=NC, num_subcores=NS)
    f = pl.kernel(
        _sc_body,
        out_type=jax.ShapeDtypeStruct((B, HIDDEN), jnp.float32),
        mesh=mesh,
        compiler_params=pltpu.CompilerParams(needs_layout_passes=False),
        scratch_types=[
            pltpu.VMEM((ROWS_PER_W,), jnp.int32),
            pltpu.VMEM((NCHUNK, CHUNK), jnp.int32),
            pltpu.VMEM((NCHUNK, CHUNK), jnp.int32),
            pltpu.VMEM((HIDDEN,), jnp.float32),
            pltpu.VMEM((HIDDEN,), jnp.float32),
            pltpu.VMEM((ROWS_PER_W,), jnp.int32),
            pltpu.VMEM((CHUNK, HIDDEN), jnp.float32),
            pltpu.VMEM((CHUNK, HIDDEN), jnp.float32),
            pltpu.VMEM((CHUNK, HIDDEN), jnp.float32),
            pltpu.VMEM((CHUNK, HIDDEN), jnp.float32),
            pltpu.VMEM((CHUNK, HIDDEN), jnp.float32),
            pltpu.VMEM((CHUNK, HIDDEN), jnp.float32),
            pltpu.SemaphoreType.DMA,
            pltpu.SemaphoreType.DMA,
            pltpu.SemaphoreType.DMA,
            pltpu.SemaphoreType.DMA,
            pltpu.SemaphoreType.DMA,
            pltpu.SemaphoreType.DMA,
        ],
    )
    return f(ids, wtab, ptt, gamma, beta)


def kernel(input_ids, word_embeddings, position_embeddings,
           token_type_embeddings, ln_gamma, ln_beta):
    ids = input_ids.reshape(-1).astype(jnp.int32)
    ptt = _fold_tt(position_embeddings, token_type_embeddings)
    out = _run(ids, word_embeddings, ptt, ln_gamma, ln_beta)
    return out.reshape(input_ids.shape + (HIDDEN,))


# speedup vs baseline: 1.9168x; 1.0049x over previous
import functools

import jax
import jax.numpy as jnp
from jax import lax
from jax.experimental import pallas as pl
from jax.experimental.pallas import tpu as pltpu
from jax.experimental.pallas import tpu_sc as plsc

HIDDEN = 768
PAD = 1
EPS = 1e-5
MAX_POS = 2050

NC, NS, L = 2, 16, 16
NW = NC * NS
B = 4 * 2048
ROWS_PER_W = B // NW
CHUNK = 16
NCHUNK = ROWS_PER_W // CHUNK
NG = HIDDEN // L


def _gs(g):
    return pl.ds(g * L, L)


def _vrsqrt(v):
    i = plsc.bitcast(v, jnp.int32)
    i = jnp.int32(0x5F3759DF) - (i >> 1)
    y = plsc.bitcast(i, jnp.float32)
    for _ in range(3):
        y = y * (1.5 - 0.5 * v * y * y)
    return y


def _fold_body(pos_ref, tt_ref, o_ref):
    o_ref[...] = pos_ref[...] + tt_ref[0:1, :]


@jax.jit
def _fold_tt(ptab, tttab):
    return pl.pallas_call(
        _fold_body,
        out_shape=jax.ShapeDtypeStruct((MAX_POS, HIDDEN), jnp.float32),
    )(ptab, tttab)


def _sc_body(ids_hbm, wtab_hbm, ptt_hbm, gamma_hbm, beta_hbm,
             out_hbm,
             ids_v, wida, pida, gam_v, bet_v, pref_v,
             wbuf0, wbuf1, pbuf0, pbuf1, obuf0, obuf1,
             wsem0, wsem1, psem0, psem1, osem0, osem1):
    c = lax.axis_index("c")
    s = lax.axis_index("s")
    wid = c * NS + s
    chk = wid % 8
    base_tok = wid * ROWS_PER_W
    seq_tok = base_tok - chk * ROWS_PER_W

    pltpu.sync_copy(ids_hbm.at[pl.ds(base_tok, ROWS_PER_W)], ids_v)
    pltpu.sync_copy(gamma_hbm, gam_v)
    pltpu.sync_copy(beta_hbm, bet_v)

    def jbody(j, acc):
        pltpu.sync_copy(ids_hbm.at[pl.ds(seq_tok + j * ROWS_PER_W,
                                         ROWS_PER_W)], pref_v)

        def cbody(i, a):
            v = pref_v[pl.ds(i * L, L)]
            return a + jnp.where(v != PAD, 1, 0).astype(jnp.int32)

        return lax.fori_loop(0, ROWS_PER_W // L, cbody, acc)

    off = lax.fori_loop(0, chk, jbody, jnp.zeros((L,), jnp.int32))
    off = jnp.full((L,), jnp.sum(off), jnp.int32)

    def pbody(i, run):
        v = ids_v[pl.ds(i * L, L)]
        m = jnp.where(v != PAD, 1, 0).astype(jnp.int32)
        cs = plsc.cumsum(m)
        wida[i] = v
        pida[i] = (cs + run) * m + PAD
        return run + jnp.sum(m)

    lax.fori_loop(0, NCHUNK, pbody, off)

    wbufs = (wbuf0, wbuf1)
    pbufs = (pbuf0, pbuf1)
    obufs = (obuf0, obuf1)
    wsems = (wsem0, wsem1)
    psems = (psem0, psem1)
    osems = (osem0, osem1)
    inv_h = jnp.float32(1.0 / HIDDEN)
    z = jnp.zeros((L,), jnp.float32)

    def chunk_step(k, b, first, last):
        wb, pb, ob = wbufs[b], pbufs[b], obufs[b]
        if not first:
            pltpu.make_async_copy(ob, out_hbm.at[pl.ds(base_tok, CHUNK)],
                                  osems[b]).wait()
        pltpu.make_async_copy(wtab_hbm.at[wida.at[k]], wb, wsems[b]).wait()
        pltpu.make_async_copy(ptt_hbm.at[pida.at[k]], pb, psems[b]).wait()

        def rbody(r, _):
            s0 = s1 = s2 = z
            q0 = q1 = q2 = z
            xs = []
            for g in range(NG):
                x = wb[r, _gs(g)] + pb[r, _gs(g)]
                xs.append(x)
                if g % 3 == 0:
                    s0 = s0 + x
                    q0 = q0 + x * x
                elif g % 3 == 1:
                    s1 = s1 + x
                    q1 = q1 + x * x
                else:
                    s2 = s2 + x
                    q2 = q2 + x * x
            sacc = s0 + s1 + s2
            qacc = q0 + q1 + q2
            mean = jnp.sum(sacc) * inv_h
            ex2 = jnp.sum(qacc) * inv_h
            mean_v = jnp.full((L,), mean, jnp.float32)
            var_v = jnp.full((L,), ex2, jnp.float32) - mean_v * mean_v + EPS
            rinv_v = _vrsqrt(var_v)
            mb_v = mean_v * rinv_v
            for g in range(NG):
                t = xs[g] * rinv_v - mb_v
                ob[r, _gs(g)] = t * gam_v[_gs(g)] + bet_v[_gs(g)]
            return 0

        lax.fori_loop(0, CHUNK, rbody, 0)
        pltpu.async_copy(ob, out_hbm.at[pl.ds(base_tok + k * CHUNK, CHUNK)],
                         osems[b])
        if not last:
            k2 = k + 2
            pltpu.async_copy(wtab_hbm.at[wida.at[k2]], wb, wsems[b])
            pltpu.async_copy(ptt_hbm.at[pida.at[k2]], pb, psems[b])

    pltpu.async_copy(wtab_hbm.at[wida.at[0]], wbuf0, wsem0)
    pltpu.async_copy(ptt_hbm.at[pida.at[0]], pbuf0, psem0)
    pltpu.async_copy(wtab_hbm.at[wida.at[1]], wbuf1, wsem1)
    pltpu.async_copy(ptt_hbm.at[pida.at[1]], pbuf1, psem1)

    chunk_step(0, 0, True, False)
    chunk_step(1, 1, True, False)

    def loop_body(i, _):
        k = 2 * i + 2
        chunk_step(k, 0, False, False)
        chunk_step(k + 1, 1, False, False)
        return 0

    lax.fori_loop(0, (NCHUNK - 4) // 2, loop_body, 0)

    chunk_step(NCHUNK - 2, 0, False, True)
    chunk_step(NCHUNK - 1, 1, False, True)

    pltpu.make_async_copy(obuf0, out_hbm.at[pl.ds(base_tok, CHUNK)],
                          osem0).wait()
    pltpu.make_async_copy(obuf1, out_hbm.at[pl.ds(base_tok, CHUNK)],
                          osem1).wait()


@jax.jit
def _run(ids, wtab, ptt, gamma, beta):
    mesh = plsc.VectorSubcoreMesh(
        core_axis_name="c", subcore_axis_name="s",
        num_cores=NC, num_subcores=NS)
    f = pl.kernel(
        _sc_body,
        out_type=jax.ShapeDtypeStruct((B, HIDDEN), jnp.float32),
        mesh=mesh,
        compiler_params=pltpu.CompilerParams(needs_layout_passes=False),
        scratch_types=[
            pltpu.VMEM((ROWS_PER_W,), jnp.int32),
            pltpu.VMEM((NCHUNK, CHUNK), jnp.int32),
            pltpu.VMEM((NCHUNK, CHUNK), jnp.int32),
            pltpu.VMEM((HIDDEN,), jnp.float32),
            pltpu.VMEM((HIDDEN,), jnp.float32),
            pltpu.VMEM((ROWS_PER_W,), jnp.int32),
            pltpu.VMEM((CHUNK, HIDDEN), jnp.float32),
            pltpu.VMEM((CHUNK, HIDDEN), jnp.float32),
            pltpu.VMEM((CHUNK, HIDDEN), jnp.float32),
            pltpu.VMEM((CHUNK, HIDDEN), jnp.float32),
            pltpu.VMEM((CHUNK, HIDDEN), jnp.float32),
            pltpu.VMEM((CHUNK, HIDDEN), jnp.float32),
            pltpu.SemaphoreType.DMA,
            pltpu.SemaphoreType.DMA,
            pltpu.SemaphoreType.DMA,
            pltpu.SemaphoreType.DMA,
            pltpu.SemaphoreType.DMA,
            pltpu.SemaphoreType.DMA,
        ],
    )
    return f(ids, wtab, ptt, gamma, beta)


def kernel(input_ids, word_embeddings, position_embeddings,
           token_type_embeddings, ln_gamma, ln_beta):
    ids = input_ids.reshape(-1).astype(jnp.int32)
    ptt = _fold_tt(position_embeddings, token_type_embeddings)
    out = _run(ids, word_embeddings, ptt, ln_gamma, ln_beta)
    return out.reshape(input_ids.shape + (HIDDEN,))
